# baseline (speedup 1.0000x reference)
import functools

import jax
import jax.numpy as jnp
import numpy as np
from jax import lax
from jax.experimental import pallas as pl
from jax.experimental.pallas import tpu as pltpu
from jax.experimental.pallas import tpu_sc as plsc
from jax.experimental.layout import Format, Layout, with_layout_constraint

_B, _L, _H = 1024, 200, 64
_N = _B * _L
_GW = 128
_BS = 16
_R = _BS * _L
_NT = 16
_EPS = 1e-12
_NW = 32
_PW = _N // _NW
_NROW = _PW // _L
_NWIN = _PW // _GW

_KM = np.zeros((2, _H), dtype=np.int32)
for _j in range(_H):
    if _j < 16:
        _KM[0, _j], _KM[1, _j] = _j << 9, 0xF << 9
    elif _j < 32:
        _KM[0, _j], _KM[1, _j] = (_j - 16) << 5, 0xF << 5
    elif _j < 48:
        _KM[0, _j], _KM[1, _j] = (_j - 32) << 1, 0xF << 1
    elif _j == 48:
        _KM[0, _j], _KM[1, _j] = 1, 1
    else:
        _KM[0, _j], _KM[1, _j] = 1 << 30, -1


def _sc_gather(table, qid, tid, did, dcid):
    mesh = plsc.VectorSubcoreMesh(core_axis_name="core", subcore_axis_name="subcore")

    @functools.partial(
        pl.kernel,
        out_type=jax.ShapeDtypeStruct((_N, 128), jnp.float32),
        mesh=mesh,
        compiler_params=pltpu.CompilerParams(use_tc_tiling_on_sc=False,
                                             needs_layout_passes=False),
        scratch_types=[
            pltpu.VMEM((_PW,), jnp.int32),
            pltpu.VMEM((_PW,), jnp.int32),
            pltpu.VMEM((_PW,), jnp.int32),
            pltpu.VMEM((_PW,), jnp.int32),
            pltpu.VMEM((_PW,), jnp.float32),
            pltpu.VMEM((_GW, _H), jnp.float32),
            pltpu.VMEM((_GW, _H), jnp.float32),
            pltpu.VMEM((_GW, 16), jnp.float32),
            pltpu.VMEM((_GW, 16), jnp.float32),
            pltpu.SemaphoreType.DMA,
            pltpu.SemaphoreType.DMA,
            pltpu.SemaphoreType.DMA,
            pltpu.SemaphoreType.DMA,
            pltpu.SemaphoreType.DMA,
            pltpu.SemaphoreType.DMA,
            pltpu.SemaphoreType.DMA,
        ],
    )
    def gather_kernel(table_hbm, qid_hbm, tid_hbm, did_hbm, dcid_hbm, out_hbm,
                      qbuf, tbuf, dbuf, cbuf, pbuf, rbuf0, rbuf1, pcol0, pcol1,
                      isem, gsem0, gsem1, wsem0, wsem1, psem0, psem1):
        wid = lax.axis_index("subcore") * 2 + lax.axis_index("core")
        base = wid * _PW
        row0 = wid * _NROW

        srcs = (qid_hbm, tid_hbm, did_hbm, dcid_hbm)
        dsts = (qbuf, tbuf, dbuf, cbuf)

        @pl.loop(0, _NROW)
        def _(r):
            off = pl.multiple_of(r * _L, 8)
            for s, d in zip(srcs, dsts):
                pltpu.async_copy(s.at[row0 + r, :], d.at[pl.ds(off, _L)], isem)

        @pl.loop(0, _NROW)
        def _(r):
            off = pl.multiple_of(r * _L, 8)
            for s, d in zip(srcs, dsts):
                pltpu.make_async_copy(s.at[row0 + r, :], d.at[pl.ds(off, _L)],
                                      isem).wait()

        @pl.loop(0, _PW // 16)
        def _(i):
            off = pl.multiple_of(i * 16, 8)
            sl = pl.ds(off, 16)
            qv = qbuf[sl]
            pv = ((tbuf[sl] << 9) | (dbuf[sl] << 5) | (cbuf[sl] << 1)
                  | jnp.where(qv == 0, 1, 0))
            pbuf[sl] = plsc.bitcast(pv, jnp.float32)

        rbufs = (rbuf0, rbuf1)
        pcols = (pcol0, pcol1)
        gsems = (gsem0, gsem1)
        wsems = (wsem0, wsem1)
        psems = (psem0, psem1)
        zer16 = jnp.zeros((16,), jnp.int32)
        lane16 = lax.broadcasted_iota(jnp.int32, (16,), 0)

        @pl.loop(0, _NWIN, step=2)
        def _(g):
            for k in (0, 1):
                gg = g + k
                woff = pl.multiple_of(gg * _GW, 8)
                r0 = base + gg * _GW

                @pl.when(gg >= 2)
                def _():
                    pltpu.make_async_copy(
                        rbufs[k],
                        out_hbm.at[pl.ds(r0 - 2 * _GW, _GW), pl.ds(0, _H)],
                        wsems[k],
                    ).wait()
                    pltpu.make_async_copy(
                        pcols[k],
                        out_hbm.at[pl.ds(r0 - 2 * _GW, _GW), pl.ds(_H, 16)],
                        psems[k],
                    ).wait()

                pltpu.async_copy(
                    table_hbm.at[qbuf.at[pl.ds(woff, _GW)]], rbufs[k], gsems[k])

                for j in range(_GW // 16):
                    poff = pl.multiple_of(gg * _GW + j * 16, 8)
                    pv = pbuf[pl.ds(poff, 16)]
                    plsc.store_scatter(pcols[k], [lane16 + j * 16, zer16], pv)

                pltpu.make_async_copy(
                    table_hbm.at[qbuf.at[pl.ds(woff, _GW)]], rbufs[k],
                    gsems[k]).wait()
                pltpu.async_copy(
                    rbufs[k],
                    out_hbm.at[pl.ds(r0, _GW), pl.ds(0, _H)], wsems[k])
                pltpu.async_copy(
                    pcols[k],
                    out_hbm.at[pl.ds(r0, _GW), pl.ds(_H, 16)], psems[k])

        for k in (0, 1):
            gg = _NWIN - 2 + k
            r0 = base + gg * _GW
            pltpu.make_async_copy(
                rbufs[k],
                out_hbm.at[pl.ds(r0, _GW), pl.ds(0, _H)], wsems[k]).wait()
            pltpu.make_async_copy(
                pcols[k],
                out_hbm.at[pl.ds(r0, _GW), pl.ds(_H, 16)], psems[k]).wait()

    return gather_kernel(table, qid, tid, did, dcid)


def _tc_body(q_ref, rel_ref, wsm_ref, wpos_ref, km_ref, gamma_ref, beta_ref,
             o_ref):
    rel = rel_ref[...].reshape(_R, _H)
    qblk = q_ref[...].reshape(_R, 128)
    pid = lax.bitcast_convert_type(qblk[:, _H:_H + 1], jnp.int32)
    kk = km_ref[0:1, :]
    mm = km_ref[1:2, :]
    oh = ((pid ^ kk) & mm) == 0
    smallm = jnp.dot(oh.astype(jnp.bfloat16), wsm_ref[...],
                     preferred_element_type=jnp.float32)
    small = smallm[:, :_H]
    m = smallm[:, _H:]
    x = qblk[:, :_H] + small + wpos_ref[...] + rel * (1.0 - m)
    s1 = jnp.sum(x, axis=-1, keepdims=True)
    s2 = jnp.sum(x * x, axis=-1, keepdims=True)
    mu = s1 * (1.0 / _H)
    var = s2 * (1.0 / _H) - mu * mu
    y = (x - mu) * lax.rsqrt(var + _EPS)
    o_ref[...] = (y * gamma_ref[...] + beta_ref[...]).reshape(_BS, _L, _H)


def kernel(question_ids, type_ids, difficulty_ids, discriminate_ids,
           question_rel_embeds, Wq, Wt, Wd, Wdisc, Wpos, ln_gamma, ln_beta):
    seq_len = question_ids.shape[1]
    idx_dtype = jnp.int64 if question_ids.dtype == jnp.int64 else jnp.int32
    position_ids = jnp.arange(seq_len, dtype=idx_dtype)[None, :]

    Wq_lin = lax.optimization_barrier(
        Wq.reshape(Wq.shape[0] * Wq.shape[1])).reshape(Wq.shape)
    qrows = _sc_gather(Wq_lin,
                       question_ids.astype(jnp.int32),
                       type_ids.astype(jnp.int32),
                       difficulty_ids.astype(jnp.int32),
                       discriminate_ids.astype(jnp.int32))

    wsm = jnp.zeros((_H, 2 * _H), dtype=jnp.bfloat16)
    wsm = wsm.at[0:_NT, :_H].set(Wt.astype(jnp.bfloat16))
    wsm = wsm.at[_NT:_NT + Wd.shape[0], :_H].set(Wd.astype(jnp.bfloat16))
    wsm = wsm.at[2 * _NT:2 * _NT + Wdisc.shape[0], :_H].set(Wdisc.astype(jnp.bfloat16))
    wsm = wsm.at[48, _H:].set(jnp.ones((_H,), jnp.bfloat16))

    wpos_t = jnp.tile(Wpos[:seq_len], (_BS, 1))
    km = jnp.asarray(_KM)
    gamma2 = ln_gamma.reshape(1, _H)
    beta2 = ln_beta.reshape(1, _H)

    x = pl.pallas_call(
        _tc_body,
        grid=(_B // _BS,),
        in_specs=[
            pl.BlockSpec((_R // 128, 128, 128), lambda i: (i, 0, 0)),
            pl.BlockSpec((_BS, _L, _H), lambda i: (i, 0, 0)),
            pl.BlockSpec((_H, 2 * _H), lambda i: (0, 0)),
            pl.BlockSpec((_R, _H), lambda i: (0, 0)),
            pl.BlockSpec((2, _H), lambda i: (0, 0)),
            pl.BlockSpec((1, _H), lambda i: (0, 0)),
            pl.BlockSpec((1, _H), lambda i: (0, 0)),
        ],
        out_specs=pl.BlockSpec((_BS, _L, _H), lambda i: (i, 0, 0)),
        out_shape=jax.ShapeDtypeStruct((_B, _L, _H), jnp.float32),
    )(qrows.reshape(_N // 128, 128, 128), question_rel_embeds, wsm, wpos_t, km,
      gamma2, beta2)

    return (x, position_ids)


# speedup vs baseline: 1.1022x; 1.1022x over previous
import functools

import jax
import jax.numpy as jnp
import numpy as np
from jax import lax
from jax.experimental import pallas as pl
from jax.experimental.pallas import tpu as pltpu
from jax.experimental.pallas import tpu_sc as plsc
from jax.experimental.layout import Format, Layout, with_layout_constraint

_B, _L, _H = 1024, 200, 64
_N = _B * _L
_GW = 128
_BS = 16
_R = _BS * _L
_NT = 16
_EPS = 1e-12
_NW = 32
_PW = _N // _NW
_NROW = _PW // _L
_NWIN = _PW // _GW

_KM = np.zeros((2, _H), dtype=np.int32)
for _j in range(_H):
    if _j < 16:
        _KM[0, _j], _KM[1, _j] = _j << 9, 0xF << 9
    elif _j < 32:
        _KM[0, _j], _KM[1, _j] = (_j - 16) << 5, 0xF << 5
    elif _j < 48:
        _KM[0, _j], _KM[1, _j] = (_j - 32) << 1, 0xF << 1
    elif _j == 48:
        _KM[0, _j], _KM[1, _j] = 1, 1
    else:
        _KM[0, _j], _KM[1, _j] = 1 << 30, -1


def _sc_gather(table, qid, tid, did, dcid):
    mesh = plsc.VectorSubcoreMesh(core_axis_name="core", subcore_axis_name="subcore")

    @functools.partial(
        pl.kernel,
        out_type=jax.ShapeDtypeStruct((_N, 128), jnp.float32),
        mesh=mesh,
        compiler_params=pltpu.CompilerParams(use_tc_tiling_on_sc=False,
                                             needs_layout_passes=False),
        scratch_types=[
            pltpu.VMEM((_PW,), jnp.int32),
            pltpu.VMEM((_PW,), jnp.int32),
            pltpu.VMEM((_PW,), jnp.int32),
            pltpu.VMEM((_PW,), jnp.int32),
            pltpu.VMEM((_PW,), jnp.float32),
            pltpu.VMEM((_GW, 128), jnp.float32),
            pltpu.VMEM((_GW, 128), jnp.float32),
            pltpu.VMEM((_GW, 16), jnp.float32),
            pltpu.VMEM((_GW, 16), jnp.float32),
            pltpu.SemaphoreType.DMA,
            pltpu.SemaphoreType.DMA,
            pltpu.SemaphoreType.DMA,
            pltpu.SemaphoreType.DMA,
            pltpu.SemaphoreType.DMA,
            pltpu.SemaphoreType.DMA,
            pltpu.SemaphoreType.DMA,
        ],
    )
    def gather_kernel(table_hbm, qid_hbm, tid_hbm, did_hbm, dcid_hbm, out_hbm,
                      qbuf, tbuf, dbuf, cbuf, pbuf, rbuf0, rbuf1, pcol0, pcol1,
                      isem, gsem0, gsem1, wsem0, wsem1, psem0, psem1):
        wid = lax.axis_index("subcore") * 2 + lax.axis_index("core")
        base = wid * _PW
        row0 = wid * _NROW

        srcs = (qid_hbm, tid_hbm, did_hbm, dcid_hbm)
        dsts = (qbuf, tbuf, dbuf, cbuf)

        @pl.loop(0, _NROW)
        def _(r):
            off = pl.multiple_of(r * _L, 8)
            for s, d in zip(srcs, dsts):
                pltpu.async_copy(s.at[row0 + r, :], d.at[pl.ds(off, _L)], isem)

        @pl.loop(0, _NROW)
        def _(r):
            off = pl.multiple_of(r * _L, 8)
            for s, d in zip(srcs, dsts):
                pltpu.make_async_copy(s.at[row0 + r, :], d.at[pl.ds(off, _L)],
                                      isem).wait()

        @pl.loop(0, _PW // 16)
        def _(i):
            off = pl.multiple_of(i * 16, 8)
            sl = pl.ds(off, 16)
            qv = qbuf[sl]
            pv = ((tbuf[sl] << 9) | (dbuf[sl] << 5) | (cbuf[sl] << 1)
                  | jnp.where(qv == 0, 1, 0))
            pbuf[sl] = plsc.bitcast(pv, jnp.float32)

        rbufs = (rbuf0, rbuf1)
        pcols = (pcol0, pcol1)
        gsems = (gsem0, gsem1)
        wsems = (wsem0, wsem1)
        psems = (psem0, psem1)
        zer16 = jnp.zeros((16,), jnp.int32)
        lane16 = lax.broadcasted_iota(jnp.int32, (16,), 0)

        @pl.loop(0, _NWIN, step=2)
        def _(g):
            for k in (0, 1):
                gg = g + k
                woff = pl.multiple_of(gg * _GW, 8)
                r0 = base + gg * _GW

                @pl.when(gg >= 2)
                def _():
                    pltpu.make_async_copy(
                        rbufs[k].at[:, pl.ds(0, _H)],
                        out_hbm.at[pl.ds(r0 - 2 * _GW, _GW), pl.ds(0, _H)],
                        wsems[k],
                    ).wait()
                    pltpu.make_async_copy(
                        pcols[k],
                        out_hbm.at[pl.ds(r0 - 2 * _GW, _GW), pl.ds(_H, 16)],
                        psems[k],
                    ).wait()

                pltpu.async_copy(
                    table_hbm.at[qbuf.at[pl.ds(woff, _GW)]], rbufs[k], gsems[k])

                for j in range(_GW // 16):
                    poff = pl.multiple_of(gg * _GW + j * 16, 8)
                    pv = pbuf[pl.ds(poff, 16)]
                    plsc.store_scatter(pcols[k], [lane16 + j * 16, zer16], pv)

                pltpu.make_async_copy(
                    table_hbm.at[qbuf.at[pl.ds(woff, _GW)]], rbufs[k],
                    gsems[k]).wait()
                pltpu.async_copy(
                    rbufs[k].at[:, pl.ds(0, _H)],
                    out_hbm.at[pl.ds(r0, _GW), pl.ds(0, _H)], wsems[k])
                pltpu.async_copy(
                    pcols[k],
                    out_hbm.at[pl.ds(r0, _GW), pl.ds(_H, 16)], psems[k])

        for k in (0, 1):
            gg = _NWIN - 2 + k
            r0 = base + gg * _GW
            pltpu.make_async_copy(
                rbufs[k].at[:, pl.ds(0, _H)],
                out_hbm.at[pl.ds(r0, _GW), pl.ds(0, _H)], wsems[k]).wait()
            pltpu.make_async_copy(
                pcols[k],
                out_hbm.at[pl.ds(r0, _GW), pl.ds(_H, 16)], psems[k]).wait()

    return gather_kernel(table, qid, tid, did, dcid)


_WB = 2048


def _repack_body(in_ref, o_ref):
    o_ref[:, :_H] = in_ref[...].T
    o_ref[:, _H:] = jnp.zeros((_WB, 128 - _H), jnp.float32)


def _wq_repack(Wq):
    nq = Wq.shape[0]
    grid = (nq + _WB - 1) // _WB
    return pl.pallas_call(
        _repack_body,
        grid=(grid,),
        in_specs=[pl.BlockSpec((_H, _WB), lambda j: (0, j))],
        out_specs=pl.BlockSpec((_WB, 128), lambda j: (j, 0)),
        out_shape=jax.ShapeDtypeStruct((nq, 128), jnp.float32),
    )(Wq.T)


def _tc_body(q_ref, rel_ref, wsm_ref, wpos_ref, km_ref, gamma_ref, beta_ref,
             o_ref):
    rel = rel_ref[...].reshape(_R, _H)
    qblk = q_ref[...].reshape(_R, 128)
    pid = lax.bitcast_convert_type(qblk[:, _H:_H + 1], jnp.int32)
    kk = km_ref[0:1, :]
    mm = km_ref[1:2, :]
    oh = ((pid ^ kk) & mm) == 0
    smallm = jnp.dot(oh.astype(jnp.bfloat16), wsm_ref[...],
                     preferred_element_type=jnp.float32)
    small = smallm[:, :_H]
    m = smallm[:, _H:]
    x = qblk[:, :_H] + small + wpos_ref[...] + rel * (1.0 - m)
    s1 = jnp.sum(x, axis=-1, keepdims=True)
    s2 = jnp.sum(x * x, axis=-1, keepdims=True)
    mu = s1 * (1.0 / _H)
    var = s2 * (1.0 / _H) - mu * mu
    y = (x - mu) * lax.rsqrt(var + _EPS)
    o_ref[...] = (y * gamma_ref[...] + beta_ref[...]).reshape(_BS, _L, _H)


def kernel(question_ids, type_ids, difficulty_ids, discriminate_ids,
           question_rel_embeds, Wq, Wt, Wd, Wdisc, Wpos, ln_gamma, ln_beta):
    seq_len = question_ids.shape[1]
    idx_dtype = jnp.int64 if question_ids.dtype == jnp.int64 else jnp.int32
    position_ids = jnp.arange(seq_len, dtype=idx_dtype)[None, :]

    Wq_lin = _wq_repack(Wq)
    qrows = _sc_gather(Wq_lin,
                       question_ids.astype(jnp.int32),
                       type_ids.astype(jnp.int32),
                       difficulty_ids.astype(jnp.int32),
                       discriminate_ids.astype(jnp.int32))

    wsm = jnp.zeros((_H, 2 * _H), dtype=jnp.bfloat16)
    wsm = wsm.at[0:_NT, :_H].set(Wt.astype(jnp.bfloat16))
    wsm = wsm.at[_NT:_NT + Wd.shape[0], :_H].set(Wd.astype(jnp.bfloat16))
    wsm = wsm.at[2 * _NT:2 * _NT + Wdisc.shape[0], :_H].set(Wdisc.astype(jnp.bfloat16))
    wsm = wsm.at[48, _H:].set(jnp.ones((_H,), jnp.bfloat16))

    wpos_t = jnp.tile(Wpos[:seq_len], (_BS, 1))
    km = jnp.asarray(_KM)
    gamma2 = ln_gamma.reshape(1, _H)
    beta2 = ln_beta.reshape(1, _H)

    x = pl.pallas_call(
        _tc_body,
        grid=(_B // _BS,),
        in_specs=[
            pl.BlockSpec((_R // 128, 128, 128), lambda i: (i, 0, 0)),
            pl.BlockSpec((_BS, _L, _H), lambda i: (i, 0, 0)),
            pl.BlockSpec((_H, 2 * _H), lambda i: (0, 0)),
            pl.BlockSpec((_R, _H), lambda i: (0, 0)),
            pl.BlockSpec((2, _H), lambda i: (0, 0)),
            pl.BlockSpec((1, _H), lambda i: (0, 0)),
            pl.BlockSpec((1, _H), lambda i: (0, 0)),
        ],
        out_specs=pl.BlockSpec((_BS, _L, _H), lambda i: (i, 0, 0)),
        out_shape=jax.ShapeDtypeStruct((_B, _L, _H), jnp.float32),
    )(qrows.reshape(_N // 128, 128, 128), question_rel_embeds, wsm, wpos_t, km,
      gamma2, beta2)

    return (x, position_ids)


# speedup vs baseline: 1.3915x; 1.2625x over previous
import functools

import jax
import jax.numpy as jnp
import numpy as np
from jax import lax
from jax.experimental import pallas as pl
from jax.experimental.pallas import tpu as pltpu
from jax.experimental.pallas import tpu_sc as plsc
from jax.experimental.layout import Format, Layout, with_layout_constraint

_B, _L, _H = 1024, 200, 64
_N = _B * _L
_GW = 128
_BS = 16
_R = _BS * _L
_NT = 16
_EPS = 1e-12
_NW = 32
_PW = _N // _NW
_NROW = _PW // _L
_NWIN = _PW // _GW

_KM = np.zeros((2, _H), dtype=np.int32)
for _j in range(_H):
    if _j < 16:
        _KM[0, _j], _KM[1, _j] = _j << 9, 0xF << 9
    elif _j < 32:
        _KM[0, _j], _KM[1, _j] = (_j - 16) << 5, 0xF << 5
    elif _j < 48:
        _KM[0, _j], _KM[1, _j] = (_j - 32) << 1, 0xF << 1
    elif _j == 48:
        _KM[0, _j], _KM[1, _j] = 1, 1
    else:
        _KM[0, _j], _KM[1, _j] = 1 << 30, -1


def _sc_gather(table, qid, tid, did, dcid):
    mesh = plsc.VectorSubcoreMesh(core_axis_name="core", subcore_axis_name="subcore")

    @functools.partial(
        pl.kernel,
        out_type=jax.ShapeDtypeStruct((_N, 128), jnp.float32),
        mesh=mesh,
        compiler_params=pltpu.CompilerParams(use_tc_tiling_on_sc=False,
                                             needs_layout_passes=False),
        scratch_types=[
            pltpu.VMEM((_PW,), jnp.int32),
            pltpu.VMEM((_PW,), jnp.int32),
            pltpu.VMEM((_PW,), jnp.int32),
            pltpu.VMEM((_PW,), jnp.int32),
            pltpu.VMEM((_PW,), jnp.float32),
            pltpu.VMEM((_GW, _H), jnp.float32),
            pltpu.VMEM((_GW, _H), jnp.float32),
            pltpu.VMEM((_GW, 16), jnp.float32),
            pltpu.VMEM((_GW, 16), jnp.float32),
            pltpu.SemaphoreType.DMA,
            pltpu.SemaphoreType.DMA,
            pltpu.SemaphoreType.DMA,
            pltpu.SemaphoreType.DMA,
            pltpu.SemaphoreType.DMA,
            pltpu.SemaphoreType.DMA,
            pltpu.SemaphoreType.DMA,
        ],
    )
    def gather_kernel(table_hbm, qid_hbm, tid_hbm, did_hbm, dcid_hbm, out_hbm,
                      qbuf, tbuf, dbuf, cbuf, pbuf, rbuf0, rbuf1, pcol0, pcol1,
                      isem, gsem0, gsem1, wsem0, wsem1, psem0, psem1):
        wid = lax.axis_index("subcore") * 2 + lax.axis_index("core")
        base = wid * _PW
        row0 = wid * _NROW

        srcs = (qid_hbm, tid_hbm, did_hbm, dcid_hbm)
        dsts = (qbuf, tbuf, dbuf, cbuf)

        @pl.loop(0, _NROW)
        def _(r):
            off = pl.multiple_of(r * _L, 8)
            for s, d in zip(srcs, dsts):
                pltpu.async_copy(s.at[row0 + r, :], d.at[pl.ds(off, _L)], isem)

        @pl.loop(0, _NROW)
        def _(r):
            off = pl.multiple_of(r * _L, 8)
            for s, d in zip(srcs, dsts):
                pltpu.make_async_copy(s.at[row0 + r, :], d.at[pl.ds(off, _L)],
                                      isem).wait()

        @pl.loop(0, _PW // 16)
        def _(i):
            off = pl.multiple_of(i * 16, 8)
            sl = pl.ds(off, 16)
            qv = qbuf[sl]
            pv = ((tbuf[sl] << 9) | (dbuf[sl] << 5) | (cbuf[sl] << 1)
                  | jnp.where(qv == 0, 1, 0))
            pbuf[sl] = plsc.bitcast(pv, jnp.float32)
            qbuf[sl] = (((qv >> 12) << 12) | ((qv & 0x7FF) << 1)
                        | ((qv >> 11) & 1))

        rbufs = (rbuf0, rbuf1)
        pcols = (pcol0, pcol1)
        gsems = (gsem0, gsem1)
        wsems = (wsem0, wsem1)
        psems = (psem0, psem1)
        zer16 = jnp.zeros((16,), jnp.int32)
        lane16 = lax.broadcasted_iota(jnp.int32, (16,), 0)

        @pl.loop(0, _NWIN, step=2)
        def _(g):
            for k in (0, 1):
                gg = g + k
                woff = pl.multiple_of(gg * _GW, 8)
                r0 = base + gg * _GW

                @pl.when(gg >= 2)
                def _():
                    pltpu.make_async_copy(
                        rbufs[k],
                        out_hbm.at[pl.ds(r0 - 2 * _GW, _GW), pl.ds(0, _H)],
                        wsems[k],
                    ).wait()
                    pltpu.make_async_copy(
                        pcols[k],
                        out_hbm.at[pl.ds(r0 - 2 * _GW, _GW), pl.ds(_H, 16)],
                        psems[k],
                    ).wait()

                pltpu.async_copy(
                    table_hbm.at[qbuf.at[pl.ds(woff, _GW)]], rbufs[k], gsems[k])

                for j in range(_GW // 16):
                    poff = pl.multiple_of(gg * _GW + j * 16, 8)
                    pv = pbuf[pl.ds(poff, 16)]
                    plsc.store_scatter(pcols[k], [lane16 + j * 16, zer16], pv)

                pltpu.make_async_copy(
                    table_hbm.at[qbuf.at[pl.ds(woff, _GW)]], rbufs[k],
                    gsems[k]).wait()
                pltpu.async_copy(
                    rbufs[k],
                    out_hbm.at[pl.ds(r0, _GW), pl.ds(0, _H)], wsems[k])
                pltpu.async_copy(
                    pcols[k],
                    out_hbm.at[pl.ds(r0, _GW), pl.ds(_H, 16)], psems[k])

        for k in (0, 1):
            gg = _NWIN - 2 + k
            r0 = base + gg * _GW
            pltpu.make_async_copy(
                rbufs[k],
                out_hbm.at[pl.ds(r0, _GW), pl.ds(0, _H)], wsems[k]).wait()
            pltpu.make_async_copy(
                pcols[k],
                out_hbm.at[pl.ds(r0, _GW), pl.ds(_H, 16)], psems[k]).wait()

    return gather_kernel(table, qid, tid, did, dcid)


_WB = 2048


def _repack_body(in_ref, o_ref):
    x = in_ref[...]
    o_ref[...] = jnp.concatenate([x[:, :_WB].T, x[:, _WB:].T], axis=1)


def _wq_repack(Wq):
    nq = Wq.shape[0]
    grid = (nq + 2 * _WB - 1) // (2 * _WB)
    return pl.pallas_call(
        _repack_body,
        grid=(grid,),
        in_specs=[pl.BlockSpec((_H, 2 * _WB), lambda j: (0, j))],
        out_specs=pl.BlockSpec((_WB, 128), lambda j: (j, 0)),
        out_shape=jax.ShapeDtypeStruct((grid * _WB, 128), jnp.float32),
    )(Wq.T)


def _tc_body(q_ref, rel_ref, wsm_ref, wpos_ref, km_ref, gamma_ref, beta_ref,
             o_ref):
    rel = rel_ref[...].reshape(_R, _H)
    qblk = q_ref[...].reshape(_R, 128)
    pid = lax.bitcast_convert_type(qblk[:, _H:_H + 1], jnp.int32)
    kk = km_ref[0:1, :]
    mm = km_ref[1:2, :]
    oh = ((pid ^ kk) & mm) == 0
    smallm = jnp.dot(oh.astype(jnp.bfloat16), wsm_ref[...],
                     preferred_element_type=jnp.float32)
    small = smallm[:, :_H]
    m = smallm[:, _H:]
    x = qblk[:, :_H] + small + wpos_ref[...] + rel * (1.0 - m)
    s1 = jnp.sum(x, axis=-1, keepdims=True)
    s2 = jnp.sum(x * x, axis=-1, keepdims=True)
    mu = s1 * (1.0 / _H)
    var = s2 * (1.0 / _H) - mu * mu
    y = (x - mu) * lax.rsqrt(var + _EPS)
    o_ref[...] = (y * gamma_ref[...] + beta_ref[...]).reshape(_BS, _L, _H)


def kernel(question_ids, type_ids, difficulty_ids, discriminate_ids,
           question_rel_embeds, Wq, Wt, Wd, Wdisc, Wpos, ln_gamma, ln_beta):
    seq_len = question_ids.shape[1]
    idx_dtype = jnp.int64 if question_ids.dtype == jnp.int64 else jnp.int32
    position_ids = jnp.arange(seq_len, dtype=idx_dtype)[None, :]

    Wq_lin = _wq_repack(Wq).reshape(-1, _H)
    qrows = _sc_gather(Wq_lin,
                       question_ids.astype(jnp.int32),
                       type_ids.astype(jnp.int32),
                       difficulty_ids.astype(jnp.int32),
                       discriminate_ids.astype(jnp.int32))

    wsm = jnp.zeros((_H, 2 * _H), dtype=jnp.bfloat16)
    wsm = wsm.at[0:_NT, :_H].set(Wt.astype(jnp.bfloat16))
    wsm = wsm.at[_NT:_NT + Wd.shape[0], :_H].set(Wd.astype(jnp.bfloat16))
    wsm = wsm.at[2 * _NT:2 * _NT + Wdisc.shape[0], :_H].set(Wdisc.astype(jnp.bfloat16))
    wsm = wsm.at[48, _H:].set(jnp.ones((_H,), jnp.bfloat16))

    wpos_t = jnp.tile(Wpos[:seq_len], (_BS, 1))
    km = jnp.asarray(_KM)
    gamma2 = ln_gamma.reshape(1, _H)
    beta2 = ln_beta.reshape(1, _H)

    x = pl.pallas_call(
        _tc_body,
        grid=(_B // _BS,),
        in_specs=[
            pl.BlockSpec((_R // 128, 128, 128), lambda i: (i, 0, 0)),
            pl.BlockSpec((_BS, _L, _H), lambda i: (i, 0, 0)),
            pl.BlockSpec((_H, 2 * _H), lambda i: (0, 0)),
            pl.BlockSpec((_R, _H), lambda i: (0, 0)),
            pl.BlockSpec((2, _H), lambda i: (0, 0)),
            pl.BlockSpec((1, _H), lambda i: (0, 0)),
            pl.BlockSpec((1, _H), lambda i: (0, 0)),
        ],
        out_specs=pl.BlockSpec((_BS, _L, _H), lambda i: (i, 0, 0)),
        out_shape=jax.ShapeDtypeStruct((_B, _L, _H), jnp.float32),
    )(qrows.reshape(_N // 128, 128, 128), question_rel_embeds, wsm, wpos_t, km,
      gamma2, beta2)

    return (x, position_ids)
